# baseline (speedup 1.0000x reference)
import functools

import numpy as np

import jax
import jax.numpy as jnp
from jax import lax
from jax.experimental import pallas as pl
from jax.experimental.pallas import tpu as pltpu
from jax.experimental.pallas import tpu_sc as plsc

_B = 128
_ROWCHUNK = 128
_CORE0_FRAC = 0.65


def _tc_pre_body(x_ref, w_ref, wp_ref, asrc_w_ref, adst_w_ref,
                 hb_ref, asrc_ref, adst_ref, amax_ref):
    h = jnp.dot(x_ref[...], w_ref[...], preferred_element_type=jnp.float32)
    hp = jnp.dot(x_ref[...], wp_ref[...], preferred_element_type=jnp.float32)
    hb_ref[...] = hp.astype(jnp.bfloat16)
    a_s = jnp.sum(h * asrc_w_ref[...], axis=1, keepdims=True)
    a_d = jnp.sum(h * adst_w_ref[...], axis=1, keepdims=True)
    asrc_ref[...] = a_s
    adst_ref[...] = a_d
    amax_ref[...] = jnp.full((1, 1), jnp.max(a_s), dtype=jnp.float32)


def _tc_merge_body(p_ref, d_ref, bias_ref, out_ref):
    n = out_ref.shape[0]
    p = p_ref[0, :n, :] + p_ref[1, :n, :]
    d = d_ref[0, :n] + d_ref[1, :n]
    out_ref[...] = p / (d[:, None] + 1e-16) + bias_ref[...][None, :]


def _sc_edge_kernel(n_chunks0, n_chunks1, rows_per_tile,
                    hb_hbm, asrc_hbm, adst_hbm, av_hbm, eidx_hbm,
                    part_hbm, den_hbm,
                    av_v,
                    e0, e1, rowsb0, rowsb1, rowsf,
                    as0, as1, ad0, ad1, w0, w1,
                    acc, dacc,
                    semg0, semg1, sema0, sema1, semd0, semd1,
                    sems, semw0, semw1):
    cid = lax.axis_index("c")
    sid = lax.axis_index("s")
    ebufs = (e0, e1)
    rbufs = (rowsb0, rowsb1)
    asbufs = (as0, as1)
    adbufs = (ad0, ad1)
    wbufs = (w0, w1)
    semgs = (semg0, semg1)
    semas = (sema0, sema1)
    semds = (semd0, semd1)
    semws = (semw0, semw1)

    pltpu.sync_copy(av_hbm, av_v)

    zeros16 = jnp.zeros((16,), jnp.float32)

    def _zero_row(j, _):
        for k in range(8):
            rowsf[j, pl.ds(k * 16, 16)] = zeros16
        return 0

    lax.fori_loop(0, _ROWCHUNK, _zero_row, 0)
    for k in range(_B // 16):
        w0[pl.ds(k * 16, 16)] = zeros16
    row_base = sid * rows_per_tile
    for r in range(rows_per_tile // _ROWCHUNK):
        pltpu.sync_copy(rowsf, acc.at[pl.ds(row_base + r * _ROWCHUNK, _ROWCHUNK)])
        pltpu.sync_copy(w0, dacc.at[pl.ds(row_base + r * _ROWCHUNK, _B)])
    plsc.subcore_barrier()

    is0 = cid == 0
    n_my = jnp.where(is0, n_chunks0, n_chunks1)
    chunk0 = jnp.where(is0, sid * n_chunks0, 16 * n_chunks0 + sid * n_chunks1)

    def _fetch(g, b):
        pltpu.sync_copy(eidx_hbm.at[pl.ds((chunk0 + g) * 2, 2)], ebufs[b])
        pltpu.async_copy(hb_hbm.at[ebufs[b].at[0]], rbufs[b], semgs[b])
        pltpu.async_copy(asrc_hbm.at[ebufs[b].at[0]], asbufs[b], semas[b])
        pltpu.async_copy(adst_hbm.at[ebufs[b].at[1]], adbufs[b], semds[b])

    _fetch(0, 0)

    def _pair(q, _):
        for b in range(2):
            g = q * 2 + b
            eb, rb, wb = ebufs[b], rbufs[b], wbufs[b]
            bn = 1 - b

            @pl.when(g >= 1)
            def _():
                pltpu.make_async_copy(
                    rowsf, acc.at[ebufs[bn].at[1]], sems).wait()

            @pl.when(g < n_my - 1)
            def _():
                @pl.when(g >= 1)
                def _():
                    pltpu.make_async_copy(
                        wbufs[bn], dacc.at[ebufs[bn].at[1]], semws[bn]).wait()
                _fetch(g + 1, bn)

            av = av_v[...]
            pltpu.make_async_copy(asrc_hbm.at[eb.at[0]], asbufs[b],
                                  semas[b]).wait()
            pltpu.make_async_copy(adst_hbm.at[eb.at[1]], adbufs[b],
                                  semds[b]).wait()
            for j in range(_B // 16):
                a_s = asbufs[b][pl.ds(j * 16, 16)]
                a_d = adbufs[b][pl.ds(j * 16, 16)]
                t = a_s + a_d
                e = jnp.where(t > 0, t, 0.2 * t)
                u = av + a_d
                m = jnp.where(u > 0, u, 0.2 * u)
                wb[pl.ds(j * 16, 16)] = jnp.exp(e - m)
            pltpu.async_copy(wb, dacc.at[eb.at[1]], semws[b], add=True)
            pltpu.make_async_copy(hb_hbm.at[eb.at[0]], rb, semgs[b]).wait()

            def _scale(j, _):
                jv = jnp.full((16,), j, dtype=jnp.int32)
                ws = plsc.load_gather(wb, [jv])
                for k in range(4):
                    hv = rb[j, pl.ds(k * 32, 32)]
                    ha, hc = plsc.unpack(hv, format=plsc.PackFormat.INTERLEAVED)
                    rowsf[j, pl.ds(k * 32, 16)] = ha * ws
                    rowsf[j, pl.ds(k * 32 + 16, 16)] = hc * ws
                return 0

            lax.fori_loop(0, _B, _scale, 0)
            pltpu.async_copy(rowsf, acc.at[eb.at[1]], sems, add=True)
        return 0

    lax.fori_loop(0, n_my // 2, _pair, 0)
    pltpu.make_async_copy(rowsf, acc.at[e0.at[1]], sems).wait()
    for b in range(2):
        pltpu.make_async_copy(wbufs[b], dacc.at[ebufs[b].at[1]],
                              semws[b]).wait()
    plsc.subcore_barrier()

    for r in range(rows_per_tile // _ROWCHUNK):
        bb = row_base + r * _ROWCHUNK
        pltpu.sync_copy(acc.at[pl.ds(bb, _ROWCHUNK)],
                        part_hbm.at[cid, pl.ds(bb, _ROWCHUNK)])
        pltpu.sync_copy(dacc.at[pl.ds(bb, _ROWCHUNK)],
                        den_hbm.at[cid, pl.ds(bb, _ROWCHUNK)])


def kernel(x, edge_index, W, att_src, att_dst, bias):
    n, in_ch = x.shape
    hidden = att_src.shape[1]
    e = edge_index.shape[1]

    np_ = ((n + 1 + 2047) // 2048) * 2048
    rows_per_tile = np_ // 16
    chunks_per_tile = 2 * (-(-e // (2 * 32 * _B)))
    ep = chunks_per_tile * 32 * _B

    x_p = jnp.pad(x, ((0, np_ - n), (0, 0)))
    src = edge_index[0].astype(jnp.int32)
    dst = edge_index[1].astype(jnp.int32)
    src_p = jnp.pad(src, (0, ep - e))
    dst_p = jnp.pad(dst, (0, ep - e), constant_values=np_ - 1)
    eidx = jnp.stack([src_p.reshape(-1, _B), dst_p.reshape(-1, _B)],
                     axis=1).reshape(-1, _B)

    perm = np.empty(hidden, dtype=np.int32)
    for m_ in range(hidden // 32):
        base = 32 * m_
        perm[base + 0:base + 32:2] = np.arange(base, base + 16)
        perm[base + 1:base + 32:2] = np.arange(base + 16, base + 32)
    W_perm = W[:, perm]

    hb, a_src2, a_dst2, amax = pl.pallas_call(
        _tc_pre_body,
        out_shape=(
            jax.ShapeDtypeStruct((np_, hidden), jnp.bfloat16),
            jax.ShapeDtypeStruct((np_, 1), jnp.float32),
            jax.ShapeDtypeStruct((np_, 1), jnp.float32),
            jax.ShapeDtypeStruct((1, 1), jnp.float32),
        ),
    )(x_p, W, W_perm, att_src[0:1, :], att_dst[0:1, :])

    a_src = a_src2.reshape(np_)
    a_dst = a_dst2.reshape(np_)
    av = jnp.broadcast_to(amax.reshape(1), (16,))

    n_pair = chunks_per_tile * 2
    n0 = 2 * int(round(_CORE0_FRAC * n_pair / 2))
    n0 = min(max(n0, 2), n_pair - 2)
    n1 = n_pair - n0

    mesh = plsc.VectorSubcoreMesh(core_axis_name="c", subcore_axis_name="s")
    sc_fn = functools.partial(_sc_edge_kernel, n0, n1, rows_per_tile)
    part, den = pl.kernel(
        sc_fn,
        mesh=mesh,
        compiler_params=pltpu.CompilerParams(needs_layout_passes=False,
                                             use_tc_tiling_on_sc=False),
        out_type=(
            jax.ShapeDtypeStruct((2, np_, hidden), jnp.float32),
            jax.ShapeDtypeStruct((2, np_), jnp.float32),
        ),
        scratch_types=[
            pltpu.VMEM((16,), jnp.float32),
            pltpu.VMEM((2, _B), jnp.int32),
            pltpu.VMEM((2, _B), jnp.int32),
            pltpu.VMEM((_B, 128), jnp.bfloat16),
            pltpu.VMEM((_B, 128), jnp.bfloat16),
            pltpu.VMEM((_B, 128), jnp.float32),
            pltpu.VMEM((_B,), jnp.float32),
            pltpu.VMEM((_B,), jnp.float32),
            pltpu.VMEM((_B,), jnp.float32),
            pltpu.VMEM((_B,), jnp.float32),
            pltpu.VMEM((_B,), jnp.float32),
            pltpu.VMEM((_B,), jnp.float32),
            pltpu.VMEM_SHARED((np_, 128), jnp.float32),
            pltpu.VMEM_SHARED((np_,), jnp.float32),
        ] + [pltpu.SemaphoreType.DMA] * 9,
    )(hb, a_src, a_dst, av, eidx)

    out = pl.pallas_call(
        _tc_merge_body,
        out_shape=jax.ShapeDtypeStruct((n, hidden), jnp.float32),
    )(part, den, bias)
    return out


# speedup vs baseline: 1.1085x; 1.1085x over previous
import functools

import numpy as np

import jax
import jax.numpy as jnp
from jax import lax
from jax.experimental import pallas as pl
from jax.experimental.pallas import tpu as pltpu
from jax.experimental.pallas import tpu_sc as plsc

_B = 128
_ROWCHUNK = 128
_CORE0_FRAC = 0.65


def _tc_pre_body(x_ref, w_ref, wp_ref, asrc_w_ref, adst_w_ref,
                 hb_ref, asrc_ref, adst_ref, amax_ref):
    h = jnp.dot(x_ref[...], w_ref[...], preferred_element_type=jnp.float32)
    hp = jnp.dot(x_ref[...], wp_ref[...], preferred_element_type=jnp.float32)
    hb_ref[...] = hp.astype(jnp.bfloat16)
    a_s = jnp.sum(h * asrc_w_ref[...], axis=1, keepdims=True)
    a_d = jnp.sum(h * adst_w_ref[...], axis=1, keepdims=True)
    asrc_ref[...] = a_s
    adst_ref[...] = a_d
    amax_ref[...] = jnp.full((1, 1), jnp.max(a_s), dtype=jnp.float32)


def _tc_merge_body(p_ref, d_ref, bias_ref, out_ref):
    n = out_ref.shape[0]
    p = p_ref[0, :n, :] + p_ref[1, :n, :]
    d = d_ref[0, :n] + d_ref[1, :n]
    out_ref[...] = p / (d[:, None] + 1e-16) + bias_ref[...][None, :]


def _sc_edge_kernel(n_chunks0, n_chunks1, rows_per_tile,
                    hb_hbm, asrc_hbm, adst_hbm, av_hbm, eidx_hbm,
                    part_hbm, den_hbm,
                    av_v,
                    e0, e1, e2, e3, rowsb0, rowsb1, rowsf,
                    as0, as1, ad0, ad1, w0, w1,
                    acc, dacc,
                    semg0, semg1, sema0, sema1, semd0, semd1,
                    sems, semw0, semw1):
    cid = lax.axis_index("c")
    sid = lax.axis_index("s")
    ebufs = (e0, e1, e2, e3)
    rbufs = (rowsb0, rowsb1)
    asbufs = (as0, as1)
    adbufs = (ad0, ad1)
    wbufs = (w0, w1)
    semgs = (semg0, semg1)
    semas = (sema0, sema1)
    semds = (semd0, semd1)
    semws = (semw0, semw1)

    pltpu.sync_copy(av_hbm, av_v)

    zeros16 = jnp.zeros((16,), jnp.float32)

    def _zero_row(j, _):
        for k in range(8):
            rowsf[j, pl.ds(k * 16, 16)] = zeros16
        return 0

    lax.fori_loop(0, _ROWCHUNK, _zero_row, 0)
    for k in range(_B // 16):
        w0[pl.ds(k * 16, 16)] = zeros16
    row_base = sid * rows_per_tile
    for r in range(rows_per_tile // _ROWCHUNK):
        pltpu.sync_copy(rowsf, acc.at[pl.ds(row_base + r * _ROWCHUNK, _ROWCHUNK)])
        pltpu.sync_copy(w0, dacc.at[pl.ds(row_base + r * _ROWCHUNK, _B)])
    plsc.subcore_barrier()

    is0 = cid == 0
    n_my = jnp.where(is0, n_chunks0, n_chunks1)
    chunk0 = jnp.where(is0, sid * n_chunks0, 16 * n_chunks0 + sid * n_chunks1)

    def _fetch(g, b4, b2):
        pltpu.sync_copy(eidx_hbm.at[pl.ds((chunk0 + g) * 2, 2)], ebufs[b4])
        pltpu.async_copy(hb_hbm.at[ebufs[b4].at[0]], rbufs[b2], semgs[b2])
        pltpu.async_copy(asrc_hbm.at[ebufs[b4].at[0]], asbufs[b2], semas[b2])
        pltpu.async_copy(adst_hbm.at[ebufs[b4].at[1]], adbufs[b2], semds[b2])

    _fetch(0, 0, 0)

    def _quad(q, _):
        for b4 in range(4):
            g = q * 4 + b4
            b2 = b4 % 2
            eb, rb, wv = ebufs[b4], rbufs[b2], wbufs[b2]

            @pl.when(g < n_my - 1)
            def _():
                _fetch(g + 1, (b4 + 1) % 4, 1 - b2)

            av = av_v[...]
            pltpu.make_async_copy(asrc_hbm.at[eb.at[0]], asbufs[b2],
                                  semas[b2]).wait()
            pltpu.make_async_copy(adst_hbm.at[eb.at[1]], adbufs[b2],
                                  semds[b2]).wait()

            @pl.when(g >= 2)
            def _():
                pltpu.make_async_copy(
                    wv, dacc.at[ebufs[(b4 + 2) % 4].at[1]], semws[b2]).wait()
            for j in range(_B // 16):
                a_s = asbufs[b2][pl.ds(j * 16, 16)]
                a_d = adbufs[b2][pl.ds(j * 16, 16)]
                t = a_s + a_d
                e = jnp.where(t > 0, t, 0.2 * t)
                u = av + a_d
                m = jnp.where(u > 0, u, 0.2 * u)
                wv[pl.ds(j * 16, 16)] = jnp.exp(e - m)
            pltpu.async_copy(wv, dacc.at[eb.at[1]], semws[b2], add=True)

            @pl.when(g >= 1)
            def _():
                pltpu.make_async_copy(
                    rowsf, acc.at[ebufs[(b4 + 3) % 4].at[1]], sems).wait()
            pltpu.make_async_copy(hb_hbm.at[eb.at[0]], rb, semgs[b2]).wait()

            def _scale(j, _):
                jv = jnp.full((16,), j, dtype=jnp.int32)
                ws = plsc.load_gather(wv, [jv])
                for k in range(4):
                    hv = rb[j, pl.ds(k * 32, 32)]
                    ha, hc = plsc.unpack(hv, format=plsc.PackFormat.INTERLEAVED)
                    rowsf[j, pl.ds(k * 32, 16)] = ha * ws
                    rowsf[j, pl.ds(k * 32 + 16, 16)] = hc * ws
                return 0

            lax.fori_loop(0, _B, _scale, 0)
            pltpu.async_copy(rowsf, acc.at[eb.at[1]], sems, add=True)
        return 0

    lax.fori_loop(0, n_my // 4, _quad, 0)
    pltpu.make_async_copy(rowsf, acc.at[e0.at[1]], sems).wait()
    for b in range(2):
        pltpu.make_async_copy(wbufs[b], dacc.at[ebufs[b].at[1]],
                              semws[b]).wait()
    plsc.subcore_barrier()

    for r in range(rows_per_tile // _ROWCHUNK):
        bb = row_base + r * _ROWCHUNK
        pltpu.sync_copy(acc.at[pl.ds(bb, _ROWCHUNK)],
                        part_hbm.at[cid, pl.ds(bb, _ROWCHUNK)])
        pltpu.sync_copy(dacc.at[pl.ds(bb, _ROWCHUNK)],
                        den_hbm.at[cid, pl.ds(bb, _ROWCHUNK)])


def kernel(x, edge_index, W, att_src, att_dst, bias):
    n, in_ch = x.shape
    hidden = att_src.shape[1]
    e = edge_index.shape[1]

    np_ = ((n + 1 + 2047) // 2048) * 2048
    rows_per_tile = np_ // 16
    chunks_per_tile = 2 * (-(-e // (2 * 32 * _B)))
    ep = chunks_per_tile * 32 * _B

    x_p = jnp.pad(x, ((0, np_ - n), (0, 0)))
    src = edge_index[0].astype(jnp.int32)
    dst = edge_index[1].astype(jnp.int32)
    src_p = jnp.pad(src, (0, ep - e))
    dst_p = jnp.pad(dst, (0, ep - e), constant_values=np_ - 1)
    eidx = jnp.stack([src_p.reshape(-1, _B), dst_p.reshape(-1, _B)],
                     axis=1).reshape(-1, _B)

    perm = np.empty(hidden, dtype=np.int32)
    for m_ in range(hidden // 32):
        base = 32 * m_
        perm[base + 0:base + 32:2] = np.arange(base, base + 16)
        perm[base + 1:base + 32:2] = np.arange(base + 16, base + 32)
    W_perm = W[:, perm]

    hb, a_src2, a_dst2, amax = pl.pallas_call(
        _tc_pre_body,
        out_shape=(
            jax.ShapeDtypeStruct((np_, hidden), jnp.bfloat16),
            jax.ShapeDtypeStruct((np_, 1), jnp.float32),
            jax.ShapeDtypeStruct((np_, 1), jnp.float32),
            jax.ShapeDtypeStruct((1, 1), jnp.float32),
        ),
    )(x_p, W, W_perm, att_src[0:1, :], att_dst[0:1, :])

    a_src = a_src2.reshape(np_)
    a_dst = a_dst2.reshape(np_)
    av = jnp.broadcast_to(amax.reshape(1), (16,))

    n_pair = chunks_per_tile * 2
    n0 = 4 * int(round(_CORE0_FRAC * n_pair / 4))
    n0 = min(max(n0, 4), n_pair - 4)
    n1 = n_pair - n0

    mesh = plsc.VectorSubcoreMesh(core_axis_name="c", subcore_axis_name="s")
    sc_fn = functools.partial(_sc_edge_kernel, n0, n1, rows_per_tile)
    part, den = pl.kernel(
        sc_fn,
        mesh=mesh,
        compiler_params=pltpu.CompilerParams(needs_layout_passes=False,
                                             use_tc_tiling_on_sc=False),
        out_type=(
            jax.ShapeDtypeStruct((2, np_, hidden), jnp.float32),
            jax.ShapeDtypeStruct((2, np_), jnp.float32),
        ),
        scratch_types=[
            pltpu.VMEM((16,), jnp.float32),
        ] + [pltpu.VMEM((2, _B), jnp.int32)] * 4 \
          + [pltpu.VMEM((_B, 128), jnp.bfloat16)] * 2 \
          + [pltpu.VMEM((_B, 128), jnp.float32)] \
          + [pltpu.VMEM((_B,), jnp.float32)] * 6 \
          + [
            pltpu.VMEM_SHARED((np_, 128), jnp.float32),
            pltpu.VMEM_SHARED((np_,), jnp.float32),
        ] + [pltpu.SemaphoreType.DMA] * 9,
    )(hb, a_src, a_dst, av, eidx)

    out = pl.pallas_call(
        _tc_merge_body,
        out_shape=jax.ShapeDtypeStruct((n, hidden), jnp.float32),
    )(part, den, bias)
    return out
